# Initial kernel scaffold; baseline (speedup 1.0000x reference)
import jax
import jax.numpy as jnp
from jax.experimental import pallas as pl


def kernel(x, edge_index, weight, W1, b1, Wmid, bmid, W9, b9):
    raise NotImplementedError("write your pallas kernel here")


# speedup vs baseline: 2.3076x; 2.3076x over previous
import functools

import jax
import jax.numpy as jnp
from jax import lax
from jax.experimental import pallas as pl
from jax.experimental.pallas import tpu as pltpu
from jax.experimental.pallas import tpu_sc as plsc

NN = 10000
EE = 320000
FIN = 128
HID = 512
COUT = 40
KHOP = 3
NMID = 7

NC = 2
NS = 16
EB = 128
N_A = 10240
BN = 256
NB = N_A // BN
STRIPE = N_A // NS
TB_HALF = 80
E_PAD = NC * NS * TB_HALF * EB
TB_FULL = E_PAD // (NS * EB)
IG = 16
NG = TB_FULL // IG

_F32 = jnp.float32
_MM_PREC = lax.Precision.HIGHEST


def _sc_mesh():
    return plsc.VectorSubcoreMesh(
        core_axis_name="c", subcore_axis_name="s",
        num_cores=NC, num_subcores=NS)


@functools.lru_cache(maxsize=None)
def _sc_prop(ncs):
    cpc = ncs // NC

    @functools.partial(
        pl.kernel,
        out_type=jax.ShapeDtypeStruct((ncs, N_A, 128), _F32),
        mesh=_sc_mesh(),
        scratch_types=[
            pltpu.VMEM((IG, EB), jnp.int32),
            pltpu.VMEM((IG, EB), jnp.int32),
            pltpu.VMEM((EB, 128), _F32),
            pltpu.VMEM((16, 128), _F32),
            pltpu.VMEM_SHARED((N_A, 128), _F32),
            pltpu.SemaphoreType.DMA,
        ],
    )
    def kern(table, row_off, col, out, rowv, colv, rowsv, zb, acc, sem):
        c = lax.axis_index("c")
        s = lax.axis_index("s")
        z16 = jnp.zeros((16,), _F32)
        for i in range(16):
            for j in range(8):
                zb[i, pl.ds(16 * j, 16)] = z16
        for i in range(cpc):
            ch = c * cpc + i
            for t in range(STRIPE // 16):
                pltpu.sync_copy(zb, acc.at[pl.ds(s * STRIPE + t * 16, 16)])
            plsc.subcore_barrier()

            def group(g, carry):
                pltpu.sync_copy(row_off.at[ch, s, pl.ds(g * IG, IG)], rowv)
                pltpu.sync_copy(col.at[s, pl.ds(g * IG, IG)], colv)

                def body(j, carry2):
                    pltpu.async_copy(table.at[rowv.at[j]], rowsv, sem).wait()
                    pltpu.sync_copy(rowsv, acc.at[colv.at[j]], add=True)
                    return carry2

                lax.fori_loop(0, IG, body, 0)
                return carry

            lax.fori_loop(0, NG, group, 0)
            plsc.subcore_barrier()
            pltpu.sync_copy(acc.at[pl.ds(s * STRIPE, STRIPE)],
                            out.at[ch, pl.ds(s * STRIPE, STRIPE)])

    return kern


@functools.lru_cache(maxsize=None)
def _sc_deg():
    @functools.partial(
        pl.kernel,
        out_type=jax.ShapeDtypeStruct((NC, N_A, 128), _F32),
        mesh=_sc_mesh(),
        scratch_types=[
            pltpu.VMEM((TB_HALF, EB), jnp.int32),
            pltpu.VMEM((EB, 128), _F32),
            pltpu.VMEM((16, 128), _F32),
            pltpu.VMEM_SHARED((N_A, 128), _F32),
        ],
    )
    def kern(col2, out, colv, onesv, zb, acc):
        c = lax.axis_index("c")
        s = lax.axis_index("s")
        z16 = jnp.zeros((16,), _F32)
        o16 = jnp.ones((16,), _F32)
        for i in range(16):
            for j in range(8):
                zb[i, pl.ds(16 * j, 16)] = z16
        for i in range(EB):
            for j in range(8):
                onesv[i, pl.ds(16 * j, 16)] = o16 if j == 0 else z16
        pltpu.sync_copy(col2.at[c, s], colv)
        for t in range(STRIPE // 16):
            pltpu.sync_copy(zb, acc.at[pl.ds(s * STRIPE + t * 16, 16)])
        plsc.subcore_barrier()

        def body(j, carry):
            pltpu.sync_copy(onesv, acc.at[colv.at[j]], add=True)
            return carry

        lax.fori_loop(0, TB_HALF, body, 0)
        plsc.subcore_barrier()
        pltpu.sync_copy(acc.at[pl.ds(s * STRIPE, STRIPE)],
                        out.at[c, pl.ds(s * STRIPE, STRIPE)])

    return kern


def _elu(x):
    return jnp.where(x > 0, x, jnp.exp(jnp.minimum(x, 0.0)) - 1.0)


def _dis_body(degp_ref, x_ref, dis_ref, dis2_ref, ax_ref):
    deg = degp_ref[0, :, 0:1] + degp_ref[1, :, 0:1]
    safe = jnp.where(deg > 0, deg, 1.0)
    d = jnp.where(deg > 0, lax.rsqrt(safe), 0.0)
    dis_ref[...] = d
    dis2_ref[...] = d * d
    ax_ref[0] = x_ref[0] * d


def _dis_call(degp, x_pad):
    return pl.pallas_call(
        _dis_body,
        grid=(NB,),
        in_specs=[
            pl.BlockSpec((NC, BN, 128), lambda i: (0, i, 0)),
            pl.BlockSpec((1, BN, 128), lambda i: (0, i, 0)),
        ],
        out_specs=[
            pl.BlockSpec((BN, 1), lambda i: (i, 0)),
            pl.BlockSpec((BN, 1), lambda i: (i, 0)),
            pl.BlockSpec((1, BN, 128), lambda i: (0, i, 0)),
        ],
        out_shape=[
            jax.ShapeDtypeStruct((N_A, 1), _F32),
            jax.ShapeDtypeStruct((N_A, 1), _F32),
            jax.ShapeDtypeStruct((1, N_A, 128), _F32),
        ],
    )(degp, x_pad)


def _scale_body(g_ref, dis2_ref, o_ref):
    o_ref[0] = g_ref[0] * dis2_ref[...]


def _scale_call(g, dis2):
    nch = g.shape[0]
    return pl.pallas_call(
        _scale_body,
        grid=(nch, NB),
        in_specs=[
            pl.BlockSpec((1, BN, 128), lambda c, i: (c, i, 0)),
            pl.BlockSpec((BN, 1), lambda c, i: (i, 0)),
        ],
        out_specs=pl.BlockSpec((1, BN, 128), lambda c, i: (c, i, 0)),
        out_shape=jax.ShapeDtypeStruct((nch, N_A, 128), _F32),
    )(g, dis2)


def _mm_factory(nci, nco, use_elu, scaled_out):
    def body(h_ref, g1_ref, g2_ref, g3_ref, dis_ref, w_ref, b_ref, *outs):
        d = dis_ref[...]
        acc = jnp.zeros((BN, 128), _F32)
        terms = ((h_ref, False), (g1_ref, True), (g2_ref, True), (g3_ref, True))
        for t, (r, scaled) in enumerate(terms):
            for k in range(nci):
                blk = r[k]
                if scaled:
                    blk = blk * d
                acc = acc + jnp.dot(blk, w_ref[0, t * nci + k],
                                    preferred_element_type=_F32,
                                    precision=_MM_PREC)
        acc = acc + b_ref[0]
        if use_elu:
            acc = _elu(acc)
        outs[0][0] = acc
        if scaled_out:
            outs[1][0] = acc * d

    out_specs = [pl.BlockSpec((1, BN, 128), lambda oc, i: (oc, i, 0))]
    out_shape = [jax.ShapeDtypeStruct((nco, N_A, 128), _F32)]
    if scaled_out:
        out_specs = out_specs * 2
        out_shape = out_shape * 2

    feat_spec = pl.BlockSpec((nci, BN, 128), lambda oc, i: (0, i, 0))
    call = pl.pallas_call(
        body,
        grid=(nco, NB),
        in_specs=[
            feat_spec, feat_spec, feat_spec, feat_spec,
            pl.BlockSpec((BN, 1), lambda oc, i: (i, 0)),
            pl.BlockSpec((1, 4 * nci, 128, 128), lambda oc, i: (oc, 0, 0, 0)),
            pl.BlockSpec((1, 1, 128), lambda oc, i: (oc, 0, 0)),
        ],
        out_specs=out_specs,
        out_shape=out_shape,
    )
    return call


def kernel(x, edge_index, weight, W1, b1, Wmid, bmid, W9, b9):
    del weight
    row = edge_index[0].astype(jnp.int32)
    col = edge_index[1].astype(jnp.int32)
    pad = E_PAD - EE
    row_p = jnp.concatenate([row, jnp.zeros((pad,), jnp.int32)])
    col_p = jnp.concatenate([col, jnp.full((pad,), NN, jnp.int32)])
    col_cs = col_p.reshape(NS, TB_FULL, EB)
    col_es = col_p.reshape(NC, NS, TB_HALF, EB)
    offs4 = (jnp.arange(4, dtype=jnp.int32) * N_A)[:, None]
    row_off4 = (row_p[None, :] + offs4).reshape(4, NS, TB_FULL, EB)
    row_off1 = jnp.broadcast_to(row_p, (2, E_PAD)).reshape(2, NS, TB_FULL, EB)

    x_pad = jnp.pad(x, ((0, N_A - NN), (0, 0))).reshape(1, N_A, 128)

    degp = _sc_deg()(col_es)
    dis, dis2, ax = _dis_call(degp, x_pad)

    def prop(table_chunks, row_off, keep):
        flat = table_chunks.reshape(-1, 128)
        return _sc_prop(row_off.shape[0])(flat, row_off, col_cs)[:keep]

    def three_hops(a0, row_off, keep):
        g1 = prop(a0, row_off, keep)
        a1 = _scale_call(g1, dis2)
        g2 = prop(a1, row_off, keep)
        a2 = _scale_call(g2, dis2)
        g3 = prop(a2, row_off, keep)
        return g1, g2, g3

    g1, g2, g3 = three_hops(ax, row_off1, 1)
    W1c = W1.reshape(4, 128, 4, 128).transpose(2, 0, 1, 3)
    b1c = b1.reshape(4, 1, 128)
    h, ah = _mm_factory(1, 4, True, True)(x_pad, g1, g2, g3, dis, W1c, b1c)

    for i in range(NMID):
        g1, g2, g3 = three_hops(ah, row_off4, 4)
        Wc = (Wmid[i].reshape(4, 4, 128, 4, 128)
              .transpose(3, 0, 1, 2, 4).reshape(4, 16, 128, 128))
        bc = bmid[i].reshape(4, 1, 128)
        h, ah = _mm_factory(4, 4, True, True)(h, g1, g2, g3, dis, Wc, bc)

    g1, g2, g3 = three_hops(ah, row_off4, 4)
    W9p = jnp.pad(W9, ((0, 0), (0, 0), (0, 128 - COUT)))
    W9c = (W9p.reshape(4, 4, 128, 1, 128)
           .transpose(3, 0, 1, 2, 4).reshape(1, 16, 128, 128))
    b9c = jnp.pad(b9, (0, 128 - COUT)).reshape(1, 1, 128)
    (out,) = _mm_factory(4, 1, False, False)(h, g1, g2, g3, dis, W9c, b9c)
    return out[0, :NN, :COUT]


# speedup vs baseline: 2.7207x; 1.1790x over previous
import functools

import jax
import jax.numpy as jnp
from jax import lax
from jax.experimental import pallas as pl
from jax.experimental.pallas import tpu as pltpu
from jax.experimental.pallas import tpu_sc as plsc

NN = 10000
EE = 320000
FIN = 128
HID = 512
COUT = 40
KHOP = 3
NMID = 7

NC = 2
NS = 16
EB = 128
N_A = 10240
BN = 256
NB = N_A // BN
STRIPE = N_A // NS
TB_HALF = 80
E_PAD = NC * NS * TB_HALF * EB
TB_FULL = E_PAD // (NS * EB)
IG = 16
NG = TB_FULL // IG

_F32 = jnp.float32
_MM_PREC = lax.Precision.HIGHEST


def _sc_mesh():
    return plsc.VectorSubcoreMesh(
        core_axis_name="c", subcore_axis_name="s",
        num_cores=NC, num_subcores=NS)


@functools.lru_cache(maxsize=None)
def _sc_prop(ncs):
    cpc = ncs // NC

    @functools.partial(
        pl.kernel,
        out_type=jax.ShapeDtypeStruct((ncs, N_A, 128), _F32),
        mesh=_sc_mesh(),
        scratch_types=[
            pltpu.VMEM((IG, EB), jnp.int32),
            pltpu.VMEM((IG, EB), jnp.int32),
            pltpu.VMEM((2, EB, 128), _F32),
            pltpu.VMEM((16, 128), _F32),
            pltpu.VMEM_SHARED((N_A, 128), _F32),
            pltpu.SemaphoreType.DMA,
        ],
    )
    def kern(table, row_off, col, out, rowv, colv, rowsv, zb, acc, sem):
        c = lax.axis_index("c")
        s = lax.axis_index("s")
        z16 = jnp.zeros((16,), _F32)
        for i in range(16):
            for j in range(8):
                zb[i, pl.ds(16 * j, 16)] = z16
        for i in range(cpc):
            ch = c * cpc + i
            for t in range(STRIPE // 16):
                pltpu.sync_copy(zb, acc.at[pl.ds(s * STRIPE + t * 16, 16)])
            plsc.subcore_barrier()

            def group(g, carry):
                pltpu.sync_copy(row_off.at[ch, s, pl.ds(g * IG, IG)], rowv)
                pltpu.sync_copy(col.at[s, pl.ds(g * IG, IG)], colv)
                descs = [
                    pltpu.async_copy(table.at[rowv.at[0]], rowsv.at[0], sem),
                    pltpu.async_copy(table.at[rowv.at[1]], rowsv.at[1], sem),
                ]
                for j in range(IG):
                    b = j & 1
                    descs[b].wait()
                    pltpu.sync_copy(rowsv.at[b], acc.at[colv.at[j]], add=True)
                    if j + 2 < IG:
                        descs[b] = pltpu.async_copy(
                            table.at[rowv.at[j + 2]], rowsv.at[b], sem)
                return carry

            lax.fori_loop(0, NG, group, 0)
            plsc.subcore_barrier()
            pltpu.sync_copy(acc.at[pl.ds(s * STRIPE, STRIPE)],
                            out.at[ch, pl.ds(s * STRIPE, STRIPE)])

    return kern


@functools.lru_cache(maxsize=None)
def _sc_deg():
    @functools.partial(
        pl.kernel,
        out_type=jax.ShapeDtypeStruct((NC, N_A, 128), _F32),
        mesh=_sc_mesh(),
        scratch_types=[
            pltpu.VMEM((TB_HALF, EB), jnp.int32),
            pltpu.VMEM((EB, 128), _F32),
            pltpu.VMEM((16, 128), _F32),
            pltpu.VMEM_SHARED((N_A, 128), _F32),
        ],
    )
    def kern(col2, out, colv, onesv, zb, acc):
        c = lax.axis_index("c")
        s = lax.axis_index("s")
        z16 = jnp.zeros((16,), _F32)
        o16 = jnp.ones((16,), _F32)
        for i in range(16):
            for j in range(8):
                zb[i, pl.ds(16 * j, 16)] = z16
        for i in range(EB):
            for j in range(8):
                onesv[i, pl.ds(16 * j, 16)] = o16 if j == 0 else z16
        pltpu.sync_copy(col2.at[c, s], colv)
        for t in range(STRIPE // 16):
            pltpu.sync_copy(zb, acc.at[pl.ds(s * STRIPE + t * 16, 16)])
        plsc.subcore_barrier()

        def body(j, carry):
            pltpu.sync_copy(onesv, acc.at[colv.at[j]], add=True)
            return carry

        lax.fori_loop(0, TB_HALF, body, 0)
        plsc.subcore_barrier()
        pltpu.sync_copy(acc.at[pl.ds(s * STRIPE, STRIPE)],
                        out.at[c, pl.ds(s * STRIPE, STRIPE)])

    return kern


def _elu(x):
    return jnp.where(x > 0, x, jnp.exp(jnp.minimum(x, 0.0)) - 1.0)


def _dis_body(degp_ref, x_ref, dis_ref, dis2_ref, ax_ref):
    deg = degp_ref[0, :, 0:1] + degp_ref[1, :, 0:1]
    safe = jnp.where(deg > 0, deg, 1.0)
    d = jnp.where(deg > 0, lax.rsqrt(safe), 0.0)
    dis_ref[...] = d
    dis2_ref[...] = d * d
    ax_ref[0] = x_ref[0] * d


def _dis_call(degp, x_pad):
    return pl.pallas_call(
        _dis_body,
        grid=(NB,),
        in_specs=[
            pl.BlockSpec((NC, BN, 128), lambda i: (0, i, 0)),
            pl.BlockSpec((1, BN, 128), lambda i: (0, i, 0)),
        ],
        out_specs=[
            pl.BlockSpec((BN, 1), lambda i: (i, 0)),
            pl.BlockSpec((BN, 1), lambda i: (i, 0)),
            pl.BlockSpec((1, BN, 128), lambda i: (0, i, 0)),
        ],
        out_shape=[
            jax.ShapeDtypeStruct((N_A, 1), _F32),
            jax.ShapeDtypeStruct((N_A, 1), _F32),
            jax.ShapeDtypeStruct((1, N_A, 128), _F32),
        ],
    )(degp, x_pad)


def _scale_body(g_ref, dis2_ref, o_ref):
    o_ref[0] = g_ref[0] * dis2_ref[...]


def _scale_call(g, dis2):
    nch = g.shape[0]
    return pl.pallas_call(
        _scale_body,
        grid=(nch, NB),
        in_specs=[
            pl.BlockSpec((1, BN, 128), lambda c, i: (c, i, 0)),
            pl.BlockSpec((BN, 1), lambda c, i: (i, 0)),
        ],
        out_specs=pl.BlockSpec((1, BN, 128), lambda c, i: (c, i, 0)),
        out_shape=jax.ShapeDtypeStruct((nch, N_A, 128), _F32),
    )(g, dis2)


def _mm_factory(nci, nco, use_elu, scaled_out):
    def body(h_ref, g1_ref, g2_ref, g3_ref, dis_ref, w_ref, b_ref, *outs):
        d = dis_ref[...]
        acc = jnp.zeros((BN, 128), _F32)
        terms = ((h_ref, False), (g1_ref, True), (g2_ref, True), (g3_ref, True))
        for t, (r, scaled) in enumerate(terms):
            for k in range(nci):
                blk = r[k]
                if scaled:
                    blk = blk * d
                acc = acc + jnp.dot(blk, w_ref[0, t * nci + k],
                                    preferred_element_type=_F32,
                                    precision=_MM_PREC)
        acc = acc + b_ref[0]
        if use_elu:
            acc = _elu(acc)
        outs[0][0] = acc
        if scaled_out:
            outs[1][0] = acc * d

    out_specs = [pl.BlockSpec((1, BN, 128), lambda oc, i: (oc, i, 0))]
    out_shape = [jax.ShapeDtypeStruct((nco, N_A, 128), _F32)]
    if scaled_out:
        out_specs = out_specs * 2
        out_shape = out_shape * 2

    feat_spec = pl.BlockSpec((nci, BN, 128), lambda oc, i: (0, i, 0))
    call = pl.pallas_call(
        body,
        grid=(nco, NB),
        in_specs=[
            feat_spec, feat_spec, feat_spec, feat_spec,
            pl.BlockSpec((BN, 1), lambda oc, i: (i, 0)),
            pl.BlockSpec((1, 4 * nci, 128, 128), lambda oc, i: (oc, 0, 0, 0)),
            pl.BlockSpec((1, 1, 128), lambda oc, i: (oc, 0, 0)),
        ],
        out_specs=out_specs,
        out_shape=out_shape,
    )
    return call


def kernel(x, edge_index, weight, W1, b1, Wmid, bmid, W9, b9):
    del weight
    row = edge_index[0].astype(jnp.int32)
    col = edge_index[1].astype(jnp.int32)
    pad = E_PAD - EE
    row_p = jnp.concatenate([row, jnp.zeros((pad,), jnp.int32)])
    col_p = jnp.concatenate([col, jnp.full((pad,), NN, jnp.int32)])
    col_cs = col_p.reshape(NS, TB_FULL, EB)
    col_es = col_p.reshape(NC, NS, TB_HALF, EB)
    offs4 = (jnp.arange(4, dtype=jnp.int32) * N_A)[:, None]
    row_off4 = (row_p[None, :] + offs4).reshape(4, NS, TB_FULL, EB)
    row_off1 = jnp.broadcast_to(row_p, (2, E_PAD)).reshape(2, NS, TB_FULL, EB)

    x_pad = jnp.pad(x, ((0, N_A - NN), (0, 0))).reshape(1, N_A, 128)

    degp = _sc_deg()(col_es)
    dis, dis2, ax = _dis_call(degp, x_pad)

    def prop(table_chunks, row_off, keep):
        flat = table_chunks.reshape(-1, 128)
        return _sc_prop(row_off.shape[0])(flat, row_off, col_cs)[:keep]

    def three_hops(a0, row_off, keep):
        g1 = prop(a0, row_off, keep)
        a1 = _scale_call(g1, dis2)
        g2 = prop(a1, row_off, keep)
        a2 = _scale_call(g2, dis2)
        g3 = prop(a2, row_off, keep)
        return g1, g2, g3

    g1, g2, g3 = three_hops(ax, row_off1, 1)
    W1c = W1.reshape(4, 128, 4, 128).transpose(2, 0, 1, 3)
    b1c = b1.reshape(4, 1, 128)
    h, ah = _mm_factory(1, 4, True, True)(x_pad, g1, g2, g3, dis, W1c, b1c)

    for i in range(NMID):
        g1, g2, g3 = three_hops(ah, row_off4, 4)
        Wc = (Wmid[i].reshape(4, 4, 128, 4, 128)
              .transpose(3, 0, 1, 2, 4).reshape(4, 16, 128, 128))
        bc = bmid[i].reshape(4, 1, 128)
        h, ah = _mm_factory(4, 4, True, True)(h, g1, g2, g3, dis, Wc, bc)

    g1, g2, g3 = three_hops(ah, row_off4, 4)
    W9p = jnp.pad(W9, ((0, 0), (0, 0), (0, 128 - COUT)))
    W9c = (W9p.reshape(4, 4, 128, 1, 128)
           .transpose(3, 0, 1, 2, 4).reshape(1, 16, 128, 128))
    b9c = jnp.pad(b9, (0, 128 - COUT)).reshape(1, 1, 128)
    (out,) = _mm_factory(4, 1, False, False)(h, g1, g2, g3, dis, W9c, b9c)
    return out[0, :NN, :COUT]


# speedup vs baseline: 2.8364x; 1.0425x over previous
import functools

import jax
import jax.numpy as jnp
from jax import lax
from jax.experimental import pallas as pl
from jax.experimental.pallas import tpu as pltpu
from jax.experimental.pallas import tpu_sc as plsc

NN = 10000
EE = 320000
FIN = 128
HID = 512
COUT = 40
KHOP = 3
NMID = 7

NC = 2
NS = 16
EB = 64
KR = 4
N_A = 10240
BN = 256
NB = N_A // BN
STRIPE = N_A // NS
E_PAD = 327680
TB_HALF = E_PAD // (NC * NS * EB)
TB_FULL = E_PAD // (NS * EB)
IG = 16
NG = TB_FULL // IG

_F32 = jnp.float32
_MM_PREC = lax.Precision.HIGHEST


def _sc_mesh():
    return plsc.VectorSubcoreMesh(
        core_axis_name="c", subcore_axis_name="s",
        num_cores=NC, num_subcores=NS)


@functools.lru_cache(maxsize=None)
def _sc_prop(ncs):
    cpc = ncs // NC

    @functools.partial(
        pl.kernel,
        out_type=jax.ShapeDtypeStruct((ncs, N_A, 128), _F32),
        mesh=_sc_mesh(),
        scratch_types=[
            pltpu.VMEM((IG, EB), jnp.int32),
            pltpu.VMEM((IG, EB), jnp.int32),
            pltpu.VMEM((KR, EB, 128), _F32),
            pltpu.VMEM((16, 128), _F32),
            pltpu.VMEM_SHARED((N_A, 128), _F32),
            pltpu.SemaphoreType.DMA,
            pltpu.SemaphoreType.DMA,
        ],
    )
    def kern(table, row_off, col, out, rowv, colv, rowsv, zb, acc, gsem, ssem):
        c = lax.axis_index("c")
        s = lax.axis_index("s")
        z16 = jnp.zeros((16,), _F32)
        for i in range(16):
            for j in range(8):
                zb[i, pl.ds(16 * j, 16)] = z16
        for i in range(cpc):
            ch = c * cpc + i
            for t in range(STRIPE // 16):
                pltpu.sync_copy(zb, acc.at[pl.ds(s * STRIPE + t * 16, 16)])
            plsc.subcore_barrier()

            def group(g, carry):
                pltpu.sync_copy(row_off.at[ch, s, pl.ds(g * IG, IG)], rowv)
                pltpu.sync_copy(col.at[s, pl.ds(g * IG, IG)], colv)
                gd = [pltpu.async_copy(table.at[rowv.at[j]], rowsv.at[j], gsem)
                      for j in range(KR)]
                sd = [None] * KR
                for j in range(IG):
                    b = j % KR
                    if j > 0:
                        bp = (j - 1) % KR
                        sd[bp].wait()
                        nj = j - 1 + KR
                        if nj < IG:
                            gd[bp] = pltpu.async_copy(
                                table.at[rowv.at[nj]], rowsv.at[bp], gsem)
                    gd[b].wait()
                    sd[b] = pltpu.async_copy(
                        rowsv.at[b], acc.at[colv.at[j]], ssem, add=True)
                sd[(IG - 1) % KR].wait()
                return carry

            lax.fori_loop(0, NG, group, 0)
            plsc.subcore_barrier()
            pltpu.sync_copy(acc.at[pl.ds(s * STRIPE, STRIPE)],
                            out.at[ch, pl.ds(s * STRIPE, STRIPE)])

    return kern


@functools.lru_cache(maxsize=None)
def _sc_deg():
    @functools.partial(
        pl.kernel,
        out_type=jax.ShapeDtypeStruct((NC, N_A, 128), _F32),
        mesh=_sc_mesh(),
        scratch_types=[
            pltpu.VMEM((TB_HALF, EB), jnp.int32),
            pltpu.VMEM((EB, 128), _F32),
            pltpu.VMEM((16, 128), _F32),
            pltpu.VMEM_SHARED((N_A, 128), _F32),
        ],
    )
    def kern(col2, out, colv, onesv, zb, acc):
        c = lax.axis_index("c")
        s = lax.axis_index("s")
        z16 = jnp.zeros((16,), _F32)
        o16 = jnp.ones((16,), _F32)
        for i in range(16):
            for j in range(8):
                zb[i, pl.ds(16 * j, 16)] = z16
        for i in range(EB):
            for j in range(8):
                onesv[i, pl.ds(16 * j, 16)] = o16 if j == 0 else z16
        pltpu.sync_copy(col2.at[c, s], colv)
        for t in range(STRIPE // 16):
            pltpu.sync_copy(zb, acc.at[pl.ds(s * STRIPE + t * 16, 16)])
        plsc.subcore_barrier()

        def body(j, carry):
            pltpu.sync_copy(onesv, acc.at[colv.at[j]], add=True)
            return carry

        lax.fori_loop(0, TB_HALF, body, 0)
        plsc.subcore_barrier()
        pltpu.sync_copy(acc.at[pl.ds(s * STRIPE, STRIPE)],
                        out.at[c, pl.ds(s * STRIPE, STRIPE)])

    return kern


def _elu(x):
    return jnp.where(x > 0, x, jnp.exp(jnp.minimum(x, 0.0)) - 1.0)


def _dis_body(degp_ref, x_ref, dis_ref, dis2_ref, ax_ref):
    deg = degp_ref[0, :, 0:1] + degp_ref[1, :, 0:1]
    safe = jnp.where(deg > 0, deg, 1.0)
    d = jnp.where(deg > 0, lax.rsqrt(safe), 0.0)
    dis_ref[...] = d
    dis2_ref[...] = d * d
    ax_ref[0] = x_ref[0] * d


def _dis_call(degp, x_pad):
    return pl.pallas_call(
        _dis_body,
        grid=(NB,),
        in_specs=[
            pl.BlockSpec((NC, BN, 128), lambda i: (0, i, 0)),
            pl.BlockSpec((1, BN, 128), lambda i: (0, i, 0)),
        ],
        out_specs=[
            pl.BlockSpec((BN, 1), lambda i: (i, 0)),
            pl.BlockSpec((BN, 1), lambda i: (i, 0)),
            pl.BlockSpec((1, BN, 128), lambda i: (0, i, 0)),
        ],
        out_shape=[
            jax.ShapeDtypeStruct((N_A, 1), _F32),
            jax.ShapeDtypeStruct((N_A, 1), _F32),
            jax.ShapeDtypeStruct((1, N_A, 128), _F32),
        ],
    )(degp, x_pad)


def _scale_body(g_ref, dis2_ref, o_ref):
    o_ref[0] = g_ref[0] * dis2_ref[...]


def _scale_call(g, dis2):
    nch = g.shape[0]
    return pl.pallas_call(
        _scale_body,
        grid=(nch, NB),
        in_specs=[
            pl.BlockSpec((1, BN, 128), lambda c, i: (c, i, 0)),
            pl.BlockSpec((BN, 1), lambda c, i: (i, 0)),
        ],
        out_specs=pl.BlockSpec((1, BN, 128), lambda c, i: (c, i, 0)),
        out_shape=jax.ShapeDtypeStruct((nch, N_A, 128), _F32),
    )(g, dis2)


def _mm_factory(nci, nco, use_elu, scaled_out):
    def body(h_ref, g1_ref, g2_ref, g3_ref, dis_ref, w_ref, b_ref, *outs):
        d = dis_ref[...]
        acc = jnp.zeros((BN, 128), _F32)
        terms = ((h_ref, False), (g1_ref, True), (g2_ref, True), (g3_ref, True))
        for t, (r, scaled) in enumerate(terms):
            for k in range(nci):
                blk = r[k]
                if scaled:
                    blk = blk * d
                acc = acc + jnp.dot(blk, w_ref[0, t * nci + k],
                                    preferred_element_type=_F32,
                                    precision=_MM_PREC)
        acc = acc + b_ref[0]
        if use_elu:
            acc = _elu(acc)
        outs[0][0] = acc
        if scaled_out:
            outs[1][0] = acc * d

    out_specs = [pl.BlockSpec((1, BN, 128), lambda oc, i: (oc, i, 0))]
    out_shape = [jax.ShapeDtypeStruct((nco, N_A, 128), _F32)]
    if scaled_out:
        out_specs = out_specs * 2
        out_shape = out_shape * 2

    feat_spec = pl.BlockSpec((nci, BN, 128), lambda oc, i: (0, i, 0))
    call = pl.pallas_call(
        body,
        grid=(nco, NB),
        in_specs=[
            feat_spec, feat_spec, feat_spec, feat_spec,
            pl.BlockSpec((BN, 1), lambda oc, i: (i, 0)),
            pl.BlockSpec((1, 4 * nci, 128, 128), lambda oc, i: (oc, 0, 0, 0)),
            pl.BlockSpec((1, 1, 128), lambda oc, i: (oc, 0, 0)),
        ],
        out_specs=out_specs,
        out_shape=out_shape,
    )
    return call


def kernel(x, edge_index, weight, W1, b1, Wmid, bmid, W9, b9):
    del weight
    row = edge_index[0].astype(jnp.int32)
    col = edge_index[1].astype(jnp.int32)
    pad = E_PAD - EE
    row_p = jnp.concatenate([row, jnp.zeros((pad,), jnp.int32)])
    col_p = jnp.concatenate([col, jnp.full((pad,), NN, jnp.int32)])
    col_cs = col_p.reshape(NS, TB_FULL, EB)
    col_es = col_p.reshape(NC, NS, TB_HALF, EB)
    offs4 = (jnp.arange(4, dtype=jnp.int32) * N_A)[:, None]
    row_off4 = (row_p[None, :] + offs4).reshape(4, NS, TB_FULL, EB)
    row_off1 = jnp.broadcast_to(row_p, (2, E_PAD)).reshape(2, NS, TB_FULL, EB)

    x_pad = jnp.pad(x, ((0, N_A - NN), (0, 0))).reshape(1, N_A, 128)

    degp = _sc_deg()(col_es)
    dis, dis2, ax = _dis_call(degp, x_pad)

    def prop(table_chunks, row_off, keep):
        flat = table_chunks.reshape(-1, 128)
        return _sc_prop(row_off.shape[0])(flat, row_off, col_cs)[:keep]

    def three_hops(a0, row_off, keep):
        g1 = prop(a0, row_off, keep)
        a1 = _scale_call(g1, dis2)
        g2 = prop(a1, row_off, keep)
        a2 = _scale_call(g2, dis2)
        g3 = prop(a2, row_off, keep)
        return g1, g2, g3

    g1, g2, g3 = three_hops(ax, row_off1, 1)
    W1c = W1.reshape(4, 128, 4, 128).transpose(2, 0, 1, 3)
    b1c = b1.reshape(4, 1, 128)
    h, ah = _mm_factory(1, 4, True, True)(x_pad, g1, g2, g3, dis, W1c, b1c)

    for i in range(NMID):
        g1, g2, g3 = three_hops(ah, row_off4, 4)
        Wc = (Wmid[i].reshape(4, 4, 128, 4, 128)
              .transpose(3, 0, 1, 2, 4).reshape(4, 16, 128, 128))
        bc = bmid[i].reshape(4, 1, 128)
        h, ah = _mm_factory(4, 4, True, True)(h, g1, g2, g3, dis, Wc, bc)

    g1, g2, g3 = three_hops(ah, row_off4, 4)
    W9p = jnp.pad(W9, ((0, 0), (0, 0), (0, 128 - COUT)))
    W9c = (W9p.reshape(4, 4, 128, 1, 128)
           .transpose(3, 0, 1, 2, 4).reshape(1, 16, 128, 128))
    b9c = jnp.pad(b9, (0, 128 - COUT)).reshape(1, 1, 128)
    (out,) = _mm_factory(4, 1, False, False)(h, g1, g2, g3, dis, W9c, b9c)
    return out[0, :NN, :COUT]


# speedup vs baseline: 3.1030x; 1.0940x over previous
import functools

import jax
import jax.numpy as jnp
from jax import lax
from jax.experimental import pallas as pl
from jax.experimental.pallas import tpu as pltpu
from jax.experimental.pallas import tpu_sc as plsc

NN = 10000
EE = 320000
FIN = 128
HID = 512
COUT = 40
KHOP = 3
NMID = 7

NC = 2
NS = 16
EB = 64
KR = 4
N_A = 10240
BN = 256
NB = N_A // BN
STRIPE = N_A // NS
E_PAD = 327680
TB_HALF = E_PAD // (NC * NS * EB)
TB_FULL = E_PAD // (NS * EB)
IG = 16
NG = TB_FULL // IG

_F32 = jnp.float32
_MM_PREC = lax.Precision.HIGHEST


def _sc_mesh():
    return plsc.VectorSubcoreMesh(
        core_axis_name="c", subcore_axis_name="s",
        num_cores=NC, num_subcores=NS)


@functools.lru_cache(maxsize=None)
def _sc_prop(ncs):
    cpc = ncs // NC

    @functools.partial(
        pl.kernel,
        out_type=jax.ShapeDtypeStruct((ncs, N_A, 128), _F32),
        mesh=_sc_mesh(),
        scratch_types=[
            pltpu.VMEM((IG, EB), jnp.int32),
            pltpu.VMEM((IG, EB), jnp.int32),
            pltpu.VMEM((KR, EB, 128), _F32),
            pltpu.VMEM((16, 128), _F32),
            pltpu.VMEM_SHARED((N_A, 128), _F32),
            pltpu.SemaphoreType.DMA,
            pltpu.SemaphoreType.DMA,
        ],
    )
    def kern(table, row_off, col, out, rowv, colv, rowsv, zb, acc, gsem, ssem):
        c = lax.axis_index("c")
        s = lax.axis_index("s")
        z16 = jnp.zeros((16,), _F32)
        for i in range(16):
            for j in range(8):
                zb[i, pl.ds(16 * j, 16)] = z16
        for i in range(cpc):
            ch = c * cpc + i
            for t in range(STRIPE // 16):
                pltpu.sync_copy(zb, acc.at[pl.ds(s * STRIPE + t * 16, 16)])
            plsc.subcore_barrier()

            def group(g, carry):
                pltpu.sync_copy(row_off.at[ch, s, pl.ds(g * IG, IG)], rowv)
                pltpu.sync_copy(col.at[s, pl.ds(g * IG, IG)], colv)
                gd = [pltpu.async_copy(table.at[rowv.at[j]], rowsv.at[j], gsem)
                      for j in range(KR)]
                sd = [None] * KR
                for j in range(IG):
                    b = j % KR
                    if j > 0:
                        bp = (j - 1) % KR
                        sd[bp].wait()
                        nj = j - 1 + KR
                        if nj < IG:
                            gd[bp] = pltpu.async_copy(
                                table.at[rowv.at[nj]], rowsv.at[bp], gsem)
                    gd[b].wait()
                    sd[b] = pltpu.async_copy(
                        rowsv.at[b], acc.at[colv.at[j]], ssem, add=True)
                sd[(IG - 1) % KR].wait()
                return carry

            lax.fori_loop(0, NG, group, 0)
            plsc.subcore_barrier()
            pltpu.sync_copy(acc.at[pl.ds(s * STRIPE, STRIPE)],
                            out.at[ch, pl.ds(s * STRIPE, STRIPE)])

    return kern


@functools.lru_cache(maxsize=None)
def _sc_prop_es():
    @functools.partial(
        pl.kernel,
        out_type=jax.ShapeDtypeStruct((NC, N_A, 128), _F32),
        mesh=_sc_mesh(),
        scratch_types=[
            pltpu.VMEM((IG, EB), jnp.int32),
            pltpu.VMEM((IG, EB), jnp.int32),
            pltpu.VMEM((KR, EB, 128), _F32),
            pltpu.VMEM((16, 128), _F32),
            pltpu.VMEM_SHARED((N_A, 128), _F32),
            pltpu.SemaphoreType.DMA,
            pltpu.SemaphoreType.DMA,
        ],
    )
    def kern(table, row2, col2, out, rowv, colv, rowsv, zb, acc, gsem, ssem):
        c = lax.axis_index("c")
        s = lax.axis_index("s")
        z16 = jnp.zeros((16,), _F32)
        for i in range(16):
            for j in range(8):
                zb[i, pl.ds(16 * j, 16)] = z16
        for t in range(STRIPE // 16):
            pltpu.sync_copy(zb, acc.at[pl.ds(s * STRIPE + t * 16, 16)])
        plsc.subcore_barrier()

        def group(g, carry):
            pltpu.sync_copy(row2.at[c, s, pl.ds(g * IG, IG)], rowv)
            pltpu.sync_copy(col2.at[c, s, pl.ds(g * IG, IG)], colv)
            gd = [pltpu.async_copy(table.at[rowv.at[j]], rowsv.at[j], gsem)
                  for j in range(KR)]
            sd = [None] * KR
            for j in range(IG):
                b = j % KR
                if j > 0:
                    bp = (j - 1) % KR
                    sd[bp].wait()
                    nj = j - 1 + KR
                    if nj < IG:
                        gd[bp] = pltpu.async_copy(
                            table.at[rowv.at[nj]], rowsv.at[bp], gsem)
                gd[b].wait()
                sd[b] = pltpu.async_copy(
                    rowsv.at[b], acc.at[colv.at[j]], ssem, add=True)
            sd[(IG - 1) % KR].wait()
            return carry

        lax.fori_loop(0, TB_HALF // IG, group, 0)
        plsc.subcore_barrier()
        pltpu.sync_copy(acc.at[pl.ds(s * STRIPE, STRIPE)],
                        out.at[c, pl.ds(s * STRIPE, STRIPE)])

    return kern


@functools.lru_cache(maxsize=None)
def _sc_deg():
    @functools.partial(
        pl.kernel,
        out_type=jax.ShapeDtypeStruct((NC, N_A, 128), _F32),
        mesh=_sc_mesh(),
        scratch_types=[
            pltpu.VMEM((TB_HALF, EB), jnp.int32),
            pltpu.VMEM((EB, 128), _F32),
            pltpu.VMEM((16, 128), _F32),
            pltpu.VMEM_SHARED((N_A, 128), _F32),
        ],
    )
    def kern(col2, out, colv, onesv, zb, acc):
        c = lax.axis_index("c")
        s = lax.axis_index("s")
        z16 = jnp.zeros((16,), _F32)
        o16 = jnp.ones((16,), _F32)
        for i in range(16):
            for j in range(8):
                zb[i, pl.ds(16 * j, 16)] = z16
        for i in range(EB):
            for j in range(8):
                onesv[i, pl.ds(16 * j, 16)] = o16 if j == 0 else z16
        pltpu.sync_copy(col2.at[c, s], colv)
        for t in range(STRIPE // 16):
            pltpu.sync_copy(zb, acc.at[pl.ds(s * STRIPE + t * 16, 16)])
        plsc.subcore_barrier()

        def body(j, carry):
            pltpu.sync_copy(onesv, acc.at[colv.at[j]], add=True)
            return carry

        lax.fori_loop(0, TB_HALF, body, 0)
        plsc.subcore_barrier()
        pltpu.sync_copy(acc.at[pl.ds(s * STRIPE, STRIPE)],
                        out.at[c, pl.ds(s * STRIPE, STRIPE)])

    return kern


def _elu(x):
    return jnp.where(x > 0, x, jnp.exp(jnp.minimum(x, 0.0)) - 1.0)


def _dis_body(degp_ref, x_ref, dis_ref, dis2_ref, ax_ref):
    deg = degp_ref[0, :, 0:1] + degp_ref[1, :, 0:1]
    safe = jnp.where(deg > 0, deg, 1.0)
    d = jnp.where(deg > 0, lax.rsqrt(safe), 0.0)
    dis_ref[...] = d
    dis2_ref[...] = d * d
    ax_ref[0] = x_ref[0] * d


def _dis_call(degp, x_pad):
    return pl.pallas_call(
        _dis_body,
        grid=(NB,),
        in_specs=[
            pl.BlockSpec((NC, BN, 128), lambda i: (0, i, 0)),
            pl.BlockSpec((1, BN, 128), lambda i: (0, i, 0)),
        ],
        out_specs=[
            pl.BlockSpec((BN, 1), lambda i: (i, 0)),
            pl.BlockSpec((BN, 1), lambda i: (i, 0)),
            pl.BlockSpec((1, BN, 128), lambda i: (0, i, 0)),
        ],
        out_shape=[
            jax.ShapeDtypeStruct((N_A, 1), _F32),
            jax.ShapeDtypeStruct((N_A, 1), _F32),
            jax.ShapeDtypeStruct((1, N_A, 128), _F32),
        ],
    )(degp, x_pad)


def _scale_body(g_ref, dis2_ref, o_ref):
    o_ref[0] = g_ref[0] * dis2_ref[...]


def _scale_call(g, dis2):
    nch = g.shape[0]
    return pl.pallas_call(
        _scale_body,
        grid=(nch, NB),
        in_specs=[
            pl.BlockSpec((1, BN, 128), lambda c, i: (c, i, 0)),
            pl.BlockSpec((BN, 1), lambda c, i: (i, 0)),
        ],
        out_specs=pl.BlockSpec((1, BN, 128), lambda c, i: (c, i, 0)),
        out_shape=jax.ShapeDtypeStruct((nch, N_A, 128), _F32),
    )(g, dis2)


def _scale2_body(g_ref, d_ref, o_ref):
    o_ref[0] = (g_ref[0] + g_ref[1]) * d_ref[...]


def _scale2_call(g, d):
    return pl.pallas_call(
        _scale2_body,
        grid=(NB,),
        in_specs=[
            pl.BlockSpec((NC, BN, 128), lambda i: (0, i, 0)),
            pl.BlockSpec((BN, 1), lambda i: (i, 0)),
        ],
        out_specs=pl.BlockSpec((1, BN, 128), lambda i: (0, i, 0)),
        out_shape=jax.ShapeDtypeStruct((1, N_A, 128), _F32),
    )(g, d)


def _hsum_body(y_ref, g_ref, dis_ref, b_ref, u_ref, t_ref):
    d = dis_ref[...]
    u = y_ref[0] + d * (g_ref[0] + g_ref[1]) + b_ref[0]
    u_ref[0] = u
    t_ref[0] = u * d


def _hsum_call(y, g, dis, b):
    return pl.pallas_call(
        _hsum_body,
        grid=(NB,),
        in_specs=[
            pl.BlockSpec((1, BN, 128), lambda i: (0, i, 0)),
            pl.BlockSpec((NC, BN, 128), lambda i: (0, i, 0)),
            pl.BlockSpec((BN, 1), lambda i: (i, 0)),
            pl.BlockSpec((1, 1, 128), lambda i: (0, 0, 0)),
        ],
        out_specs=[pl.BlockSpec((1, BN, 128), lambda i: (0, i, 0))] * 2,
        out_shape=[jax.ShapeDtypeStruct((1, N_A, 128), _F32)] * 2,
    )(y, g, dis, b)


def _mm_plain_factory(nci, nco):
    def body(h_ref, w_ref, out_ref):
        acc = jnp.zeros((BN, 128), _F32)
        for k in range(nci):
            acc = acc + jnp.dot(h_ref[k], w_ref[0, k],
                                preferred_element_type=_F32,
                                precision=_MM_PREC)
        out_ref[0] = acc

    return pl.pallas_call(
        body,
        grid=(nco, NB),
        in_specs=[
            pl.BlockSpec((nci, BN, 128), lambda oc, i: (0, i, 0)),
            pl.BlockSpec((1, nci, 128, 128), lambda oc, i: (oc, 0, 0, 0)),
        ],
        out_specs=pl.BlockSpec((1, BN, 128), lambda oc, i: (oc, i, 0)),
        out_shape=jax.ShapeDtypeStruct((nco, N_A, 128), _F32),
    )


def _mm_factory(nci, nco, use_elu, scaled_out):
    def body(h_ref, g1_ref, g2_ref, g3_ref, dis_ref, w_ref, b_ref, *outs):
        d = dis_ref[...]
        acc = jnp.zeros((BN, 128), _F32)
        terms = ((h_ref, False), (g1_ref, True), (g2_ref, True), (g3_ref, True))
        for t, (r, scaled) in enumerate(terms):
            for k in range(nci):
                blk = r[k]
                if scaled:
                    blk = blk * d
                acc = acc + jnp.dot(blk, w_ref[0, t * nci + k],
                                    preferred_element_type=_F32,
                                    precision=_MM_PREC)
        acc = acc + b_ref[0]
        if use_elu:
            acc = _elu(acc)
        outs[0][0] = acc
        if scaled_out:
            outs[1][0] = acc * d

    out_specs = [pl.BlockSpec((1, BN, 128), lambda oc, i: (oc, i, 0))]
    out_shape = [jax.ShapeDtypeStruct((nco, N_A, 128), _F32)]
    if scaled_out:
        out_specs = out_specs * 2
        out_shape = out_shape * 2

    feat_spec = pl.BlockSpec((nci, BN, 128), lambda oc, i: (0, i, 0))
    call = pl.pallas_call(
        body,
        grid=(nco, NB),
        in_specs=[
            feat_spec, feat_spec, feat_spec, feat_spec,
            pl.BlockSpec((BN, 1), lambda oc, i: (i, 0)),
            pl.BlockSpec((1, 4 * nci, 128, 128), lambda oc, i: (oc, 0, 0, 0)),
            pl.BlockSpec((1, 1, 128), lambda oc, i: (oc, 0, 0)),
        ],
        out_specs=out_specs,
        out_shape=out_shape,
    )
    return call


def kernel(x, edge_index, weight, W1, b1, Wmid, bmid, W9, b9):
    del weight
    row = edge_index[0].astype(jnp.int32)
    col = edge_index[1].astype(jnp.int32)
    pad = E_PAD - EE
    row_p = jnp.concatenate([row, jnp.zeros((pad,), jnp.int32)])
    col_p = jnp.concatenate([col, jnp.full((pad,), NN, jnp.int32)])
    col_cs = col_p.reshape(NS, TB_FULL, EB)
    col_es = col_p.reshape(NC, NS, TB_HALF, EB)
    row_es = row_p.reshape(NC, NS, TB_HALF, EB)
    offs4 = (jnp.arange(4, dtype=jnp.int32) * N_A)[:, None]
    row_off4 = (row_p[None, :] + offs4).reshape(4, NS, TB_FULL, EB)

    x_pad = jnp.pad(x, ((0, N_A - NN), (0, 0))).reshape(1, N_A, 128)

    degp = _sc_deg()(col_es)
    dis, dis2, ax = _dis_call(degp, x_pad)

    def prop(table_chunks, row_off, keep):
        flat = table_chunks.reshape(-1, 128)
        return _sc_prop(row_off.shape[0])(flat, row_off, col_cs)[:keep]

    def three_hops(a0, row_off, keep):
        g1 = prop(a0, row_off, keep)
        a1 = _scale_call(g1, dis2)
        g2 = prop(a1, row_off, keep)
        a2 = _scale_call(g2, dis2)
        g3 = prop(a2, row_off, keep)
        return g1, g2, g3

    def prop_es(table_slab):
        return _sc_prop_es()(table_slab.reshape(N_A, 128), row_es, col_es)

    g1 = prop_es(ax)
    a1 = _scale2_call(g1, dis2)
    g2 = prop_es(a1)
    a2 = _scale2_call(g2, dis2)
    g3 = prop_es(a2)
    x2 = jnp.concatenate([x_pad, jnp.zeros_like(x_pad)], axis=0)
    base = W1.reshape(4, 128, 4, 128).transpose(2, 0, 1, 3)
    Wd = jnp.stack([base, base], axis=2).at[:, 0, 1].set(0.0)
    W1c = Wd.reshape(4, 8, 128, 128)
    b1c = b1.reshape(4, 1, 128)
    h, ah = _mm_factory(2, 4, True, True)(x2, g1, g2, g3, dis, W1c, b1c)

    for i in range(NMID):
        g1, g2, g3 = three_hops(ah, row_off4, 4)
        Wc = (Wmid[i].reshape(4, 4, 128, 4, 128)
              .transpose(3, 0, 1, 2, 4).reshape(4, 16, 128, 128))
        bc = bmid[i].reshape(4, 1, 128)
        h, ah = _mm_factory(4, 4, True, True)(h, g1, g2, g3, dis, Wc, bc)

    W9p = jnp.pad(W9, ((0, 0), (0, 0), (0, 128 - COUT)))
    W9c = W9p.reshape(4, 4, 128, 128)
    yc = _mm_plain_factory(4, 4)(h, W9c)
    zb128 = jnp.zeros((1, 1, 128), _F32)
    b9c = jnp.pad(b9, (0, 128 - COUT)).reshape(1, 1, 128)
    t = _scale_call(yc[3:4], dis)
    u, t = _hsum_call(yc[2:3], prop_es(t), dis, zb128)
    u, t = _hsum_call(yc[1:2], prop_es(t), dis, zb128)
    u, _ = _hsum_call(yc[0:1], prop_es(t), dis, b9c)
    return u[0, :NN, :COUT]


# speedup vs baseline: 3.1751x; 1.0232x over previous
import functools

import jax
import jax.numpy as jnp
from jax import lax
from jax.experimental import pallas as pl
from jax.experimental.pallas import tpu as pltpu
from jax.experimental.pallas import tpu_sc as plsc

NN = 10000
EE = 320000
FIN = 128
HID = 512
COUT = 40
KHOP = 3
NMID = 7

NC = 2
NS = 16
EB = 64
KR = 4
N_A = 10240
BN = 256
NB = N_A // BN
STRIPE = N_A // NS
E_PAD = 327680
TB_HALF = E_PAD // (NC * NS * EB)
TB_FULL = E_PAD // (NS * EB)
IG = 16
NG = TB_FULL // IG

_F32 = jnp.float32
_MM_PREC = lax.Precision.HIGHEST


def _sc_mesh():
    return plsc.VectorSubcoreMesh(
        core_axis_name="c", subcore_axis_name="s",
        num_cores=NC, num_subcores=NS)


@functools.lru_cache(maxsize=None)
def _sc_prop(ncs):
    cpc = ncs // NC

    @functools.partial(
        pl.kernel,
        out_type=jax.ShapeDtypeStruct((ncs, N_A, 128), _F32),
        mesh=_sc_mesh(),
        scratch_types=[
            pltpu.VMEM((2 * IG, EB), jnp.int32),
            pltpu.VMEM((2 * IG, EB), jnp.int32),
            pltpu.VMEM((KR, EB, 128), _F32),
            pltpu.VMEM((16, 128), _F32),
            pltpu.VMEM_SHARED((N_A, 128), _F32),
            pltpu.SemaphoreType.DMA,
            pltpu.SemaphoreType.DMA,
            pltpu.SemaphoreType.DMA,
        ],
    )
    def kern(table, row_off, col, out, rowv, colv, rowsv, zb, acc,
             gsem, ssem, isem):
        c = lax.axis_index("c")
        s = lax.axis_index("s")
        z16 = jnp.zeros((16,), _F32)
        for i in range(16):
            for j in range(8):
                zb[i, pl.ds(16 * j, 16)] = z16

        def _stage(g, b):
            pltpu.async_copy(row_off.at[ch, s, pl.ds(g * IG, IG)],
                             rowv.at[pl.ds(b * IG, IG)], isem)
            pltpu.async_copy(col.at[s, pl.ds(g * IG, IG)],
                             colv.at[pl.ds(b * IG, IG)], isem)

        def _stage_wait():
            pltpu.make_async_copy(row_off.at[0, 0, pl.ds(0, IG)],
                                  rowv.at[pl.ds(0, IG)], isem).wait()
            pltpu.make_async_copy(col.at[0, pl.ds(0, IG)],
                                  colv.at[pl.ds(0, IG)], isem).wait()

        for i in range(cpc):
            ch = c * cpc + i
            for t in range(STRIPE // 16):
                pltpu.sync_copy(zb, acc.at[pl.ds(s * STRIPE + t * 16, 16)])
            _stage(0, 0)
            plsc.subcore_barrier()

            def group(g, carry):
                b = lax.rem(g, 2)
                boff = b * IG
                _stage_wait()

                @pl.when(g + 1 < NG)
                def _():
                    _stage(g + 1, 1 - b)

                gd = [pltpu.async_copy(table.at[rowv.at[boff + j]],
                                       rowsv.at[j], gsem)
                      for j in range(KR)]
                sd = [None] * KR
                for j in range(IG):
                    rb = j % KR
                    if j > 0:
                        bp = (j - 1) % KR
                        sd[bp].wait()
                        nj = j - 1 + KR
                        if nj < IG:
                            gd[bp] = pltpu.async_copy(
                                table.at[rowv.at[boff + nj]],
                                rowsv.at[bp], gsem)
                    gd[rb].wait()
                    sd[rb] = pltpu.async_copy(
                        rowsv.at[rb], acc.at[colv.at[boff + j]],
                        ssem, add=True)
                sd[(IG - 1) % KR].wait()
                return carry

            lax.fori_loop(0, NG, group, 0)
            plsc.subcore_barrier()
            pltpu.sync_copy(acc.at[pl.ds(s * STRIPE, STRIPE)],
                            out.at[ch, pl.ds(s * STRIPE, STRIPE)])

    return kern


@functools.lru_cache(maxsize=None)
def _sc_prop_es():
    @functools.partial(
        pl.kernel,
        out_type=jax.ShapeDtypeStruct((NC, N_A, 128), _F32),
        mesh=_sc_mesh(),
        scratch_types=[
            pltpu.VMEM((IG, EB), jnp.int32),
            pltpu.VMEM((IG, EB), jnp.int32),
            pltpu.VMEM((KR, EB, 128), _F32),
            pltpu.VMEM((16, 128), _F32),
            pltpu.VMEM_SHARED((N_A, 128), _F32),
            pltpu.SemaphoreType.DMA,
            pltpu.SemaphoreType.DMA,
        ],
    )
    def kern(table, row2, col2, out, rowv, colv, rowsv, zb, acc, gsem, ssem):
        c = lax.axis_index("c")
        s = lax.axis_index("s")
        z16 = jnp.zeros((16,), _F32)
        for i in range(16):
            for j in range(8):
                zb[i, pl.ds(16 * j, 16)] = z16
        for t in range(STRIPE // 16):
            pltpu.sync_copy(zb, acc.at[pl.ds(s * STRIPE + t * 16, 16)])
        plsc.subcore_barrier()

        def group(g, carry):
            pltpu.sync_copy(row2.at[c, s, pl.ds(g * IG, IG)], rowv)
            pltpu.sync_copy(col2.at[c, s, pl.ds(g * IG, IG)], colv)
            gd = [pltpu.async_copy(table.at[rowv.at[j]], rowsv.at[j], gsem)
                  for j in range(KR)]
            sd = [None] * KR
            for j in range(IG):
                b = j % KR
                if j > 0:
                    bp = (j - 1) % KR
                    sd[bp].wait()
                    nj = j - 1 + KR
                    if nj < IG:
                        gd[bp] = pltpu.async_copy(
                            table.at[rowv.at[nj]], rowsv.at[bp], gsem)
                gd[b].wait()
                sd[b] = pltpu.async_copy(
                    rowsv.at[b], acc.at[colv.at[j]], ssem, add=True)
            sd[(IG - 1) % KR].wait()
            return carry

        lax.fori_loop(0, TB_HALF // IG, group, 0)
        plsc.subcore_barrier()
        pltpu.sync_copy(acc.at[pl.ds(s * STRIPE, STRIPE)],
                        out.at[c, pl.ds(s * STRIPE, STRIPE)])

    return kern


@functools.lru_cache(maxsize=None)
def _sc_deg():
    @functools.partial(
        pl.kernel,
        out_type=jax.ShapeDtypeStruct((NC, N_A, 128), _F32),
        mesh=_sc_mesh(),
        scratch_types=[
            pltpu.VMEM((TB_HALF, EB), jnp.int32),
            pltpu.VMEM((EB, 128), _F32),
            pltpu.VMEM((16, 128), _F32),
            pltpu.VMEM_SHARED((N_A, 128), _F32),
        ],
    )
    def kern(col2, out, colv, onesv, zb, acc):
        c = lax.axis_index("c")
        s = lax.axis_index("s")
        z16 = jnp.zeros((16,), _F32)
        o16 = jnp.ones((16,), _F32)
        for i in range(16):
            for j in range(8):
                zb[i, pl.ds(16 * j, 16)] = z16
        for i in range(EB):
            for j in range(8):
                onesv[i, pl.ds(16 * j, 16)] = o16 if j == 0 else z16
        pltpu.sync_copy(col2.at[c, s], colv)
        for t in range(STRIPE // 16):
            pltpu.sync_copy(zb, acc.at[pl.ds(s * STRIPE + t * 16, 16)])
        plsc.subcore_barrier()

        def body(j, carry):
            pltpu.sync_copy(onesv, acc.at[colv.at[j]], add=True)
            return carry

        lax.fori_loop(0, TB_HALF, body, 0)
        plsc.subcore_barrier()
        pltpu.sync_copy(acc.at[pl.ds(s * STRIPE, STRIPE)],
                        out.at[c, pl.ds(s * STRIPE, STRIPE)])

    return kern


def _elu(x):
    return jnp.where(x > 0, x, jnp.exp(jnp.minimum(x, 0.0)) - 1.0)


def _dis_body(degp_ref, x_ref, dis_ref, dis2_ref, ax_ref):
    deg = degp_ref[0, :, 0:1] + degp_ref[1, :, 0:1]
    safe = jnp.where(deg > 0, deg, 1.0)
    d = jnp.where(deg > 0, lax.rsqrt(safe), 0.0)
    dis_ref[...] = d
    dis2_ref[...] = d * d
    ax_ref[0] = x_ref[0] * d


def _dis_call(degp, x_pad):
    return pl.pallas_call(
        _dis_body,
        grid=(NB,),
        in_specs=[
            pl.BlockSpec((NC, BN, 128), lambda i: (0, i, 0)),
            pl.BlockSpec((1, BN, 128), lambda i: (0, i, 0)),
        ],
        out_specs=[
            pl.BlockSpec((BN, 1), lambda i: (i, 0)),
            pl.BlockSpec((BN, 1), lambda i: (i, 0)),
            pl.BlockSpec((1, BN, 128), lambda i: (0, i, 0)),
        ],
        out_shape=[
            jax.ShapeDtypeStruct((N_A, 1), _F32),
            jax.ShapeDtypeStruct((N_A, 1), _F32),
            jax.ShapeDtypeStruct((1, N_A, 128), _F32),
        ],
    )(degp, x_pad)


def _scale_body(g_ref, dis2_ref, o_ref):
    o_ref[0] = g_ref[0] * dis2_ref[...]


def _scale_call(g, dis2):
    nch = g.shape[0]
    return pl.pallas_call(
        _scale_body,
        grid=(nch, NB),
        in_specs=[
            pl.BlockSpec((1, BN, 128), lambda c, i: (c, i, 0)),
            pl.BlockSpec((BN, 1), lambda c, i: (i, 0)),
        ],
        out_specs=pl.BlockSpec((1, BN, 128), lambda c, i: (c, i, 0)),
        out_shape=jax.ShapeDtypeStruct((nch, N_A, 128), _F32),
    )(g, dis2)


def _scale2_body(g_ref, d_ref, o_ref):
    o_ref[0] = (g_ref[0] + g_ref[1]) * d_ref[...]


def _scale2_call(g, d):
    return pl.pallas_call(
        _scale2_body,
        grid=(NB,),
        in_specs=[
            pl.BlockSpec((NC, BN, 128), lambda i: (0, i, 0)),
            pl.BlockSpec((BN, 1), lambda i: (i, 0)),
        ],
        out_specs=pl.BlockSpec((1, BN, 128), lambda i: (0, i, 0)),
        out_shape=jax.ShapeDtypeStruct((1, N_A, 128), _F32),
    )(g, d)


def _hsum_body(y_ref, g_ref, dis_ref, b_ref, u_ref, t_ref):
    d = dis_ref[...]
    u = y_ref[0] + d * (g_ref[0] + g_ref[1]) + b_ref[0]
    u_ref[0] = u
    t_ref[0] = u * d


def _hsum_call(y, g, dis, b):
    return pl.pallas_call(
        _hsum_body,
        grid=(NB,),
        in_specs=[
            pl.BlockSpec((1, BN, 128), lambda i: (0, i, 0)),
            pl.BlockSpec((NC, BN, 128), lambda i: (0, i, 0)),
            pl.BlockSpec((BN, 1), lambda i: (i, 0)),
            pl.BlockSpec((1, 1, 128), lambda i: (0, 0, 0)),
        ],
        out_specs=[pl.BlockSpec((1, BN, 128), lambda i: (0, i, 0))] * 2,
        out_shape=[jax.ShapeDtypeStruct((1, N_A, 128), _F32)] * 2,
    )(y, g, dis, b)


def _mm_plain_factory(nci, nco):
    def body(h_ref, w_ref, out_ref):
        acc = jnp.zeros((BN, 128), _F32)
        for k in range(nci):
            acc = acc + jnp.dot(h_ref[k], w_ref[0, k],
                                preferred_element_type=_F32,
                                precision=_MM_PREC)
        out_ref[0] = acc

    return pl.pallas_call(
        body,
        grid=(nco, NB),
        in_specs=[
            pl.BlockSpec((nci, BN, 128), lambda oc, i: (0, i, 0)),
            pl.BlockSpec((1, nci, 128, 128), lambda oc, i: (oc, 0, 0, 0)),
        ],
        out_specs=pl.BlockSpec((1, BN, 128), lambda oc, i: (oc, i, 0)),
        out_shape=jax.ShapeDtypeStruct((nco, N_A, 128), _F32),
    )


def _mm_factory(nci, nco, use_elu, scaled_out):
    def body(h_ref, g1_ref, g2_ref, g3_ref, dis_ref, w_ref, b_ref, *outs):
        d = dis_ref[...]
        acc = jnp.zeros((BN, 128), _F32)
        terms = ((h_ref, False), (g1_ref, True), (g2_ref, True), (g3_ref, True))
        for t, (r, scaled) in enumerate(terms):
            for k in range(nci):
                blk = r[k]
                if scaled:
                    blk = blk * d
                acc = acc + jnp.dot(blk, w_ref[0, t * nci + k],
                                    preferred_element_type=_F32,
                                    precision=_MM_PREC)
        acc = acc + b_ref[0]
        if use_elu:
            acc = _elu(acc)
        outs[0][0] = acc
        if scaled_out:
            outs[1][0] = acc * d

    out_specs = [pl.BlockSpec((1, BN, 128), lambda oc, i: (oc, i, 0))]
    out_shape = [jax.ShapeDtypeStruct((nco, N_A, 128), _F32)]
    if scaled_out:
        out_specs = out_specs * 2
        out_shape = out_shape * 2

    feat_spec = pl.BlockSpec((nci, BN, 128), lambda oc, i: (0, i, 0))
    call = pl.pallas_call(
        body,
        grid=(nco, NB),
        in_specs=[
            feat_spec, feat_spec, feat_spec, feat_spec,
            pl.BlockSpec((BN, 1), lambda oc, i: (i, 0)),
            pl.BlockSpec((1, 4 * nci, 128, 128), lambda oc, i: (oc, 0, 0, 0)),
            pl.BlockSpec((1, 1, 128), lambda oc, i: (oc, 0, 0)),
        ],
        out_specs=out_specs,
        out_shape=out_shape,
    )
    return call


def kernel(x, edge_index, weight, W1, b1, Wmid, bmid, W9, b9):
    del weight
    row = edge_index[0].astype(jnp.int32)
    col = edge_index[1].astype(jnp.int32)
    pad = E_PAD - EE
    row_p = jnp.concatenate([row, jnp.zeros((pad,), jnp.int32)])
    col_p = jnp.concatenate([col, jnp.full((pad,), NN, jnp.int32)])
    col_cs = col_p.reshape(NS, TB_FULL, EB)
    col_es = col_p.reshape(NC, NS, TB_HALF, EB)
    row_es = row_p.reshape(NC, NS, TB_HALF, EB)
    offs4 = (jnp.arange(4, dtype=jnp.int32) * N_A)[:, None]
    row_off4 = (row_p[None, :] + offs4).reshape(4, NS, TB_FULL, EB)

    x_pad = jnp.pad(x, ((0, N_A - NN), (0, 0))).reshape(1, N_A, 128)

    degp = _sc_deg()(col_es)
    dis, dis2, ax = _dis_call(degp, x_pad)

    def prop(table_chunks, row_off, keep):
        flat = table_chunks.reshape(-1, 128)
        return _sc_prop(row_off.shape[0])(flat, row_off, col_cs)[:keep]

    def three_hops(a0, row_off, keep):
        g1 = prop(a0, row_off, keep)
        a1 = _scale_call(g1, dis2)
        g2 = prop(a1, row_off, keep)
        a2 = _scale_call(g2, dis2)
        g3 = prop(a2, row_off, keep)
        return g1, g2, g3

    def prop_es(table_slab):
        return _sc_prop_es()(table_slab.reshape(N_A, 128), row_es, col_es)

    g1 = prop_es(ax)
    a1 = _scale2_call(g1, dis2)
    g2 = prop_es(a1)
    a2 = _scale2_call(g2, dis2)
    g3 = prop_es(a2)
    x2 = jnp.concatenate([x_pad, jnp.zeros_like(x_pad)], axis=0)
    base = W1.reshape(4, 128, 4, 128).transpose(2, 0, 1, 3)
    Wd = jnp.stack([base, base], axis=2).at[:, 0, 1].set(0.0)
    W1c = Wd.reshape(4, 8, 128, 128)
    b1c = b1.reshape(4, 1, 128)
    h, ah = _mm_factory(2, 4, True, True)(x2, g1, g2, g3, dis, W1c, b1c)

    for i in range(NMID):
        g1, g2, g3 = three_hops(ah, row_off4, 4)
        Wc = (Wmid[i].reshape(4, 4, 128, 4, 128)
              .transpose(3, 0, 1, 2, 4).reshape(4, 16, 128, 128))
        bc = bmid[i].reshape(4, 1, 128)
        h, ah = _mm_factory(4, 4, True, True)(h, g1, g2, g3, dis, Wc, bc)

    W9p = jnp.pad(W9, ((0, 0), (0, 0), (0, 128 - COUT)))
    W9c = W9p.reshape(4, 4, 128, 128)
    yc = _mm_plain_factory(4, 4)(h, W9c)
    zb128 = jnp.zeros((1, 1, 128), _F32)
    b9c = jnp.pad(b9, (0, 128 - COUT)).reshape(1, 1, 128)
    t = _scale_call(yc[3:4], dis)
    u, t = _hsum_call(yc[2:3], prop_es(t), dis, zb128)
    u, t = _hsum_call(yc[1:2], prop_es(t), dis, zb128)
    u, _ = _hsum_call(yc[0:1], prop_es(t), dis, b9c)
    return u[0, :NN, :COUT]
